# baseline (speedup 1.0000x reference)
import functools

import jax
import jax.numpy as jnp
from jax import lax
from jax.experimental import pallas as pl
from jax.experimental.pallas import tpu as pltpu
from jax.experimental.pallas import tpu_sc as plsc

V = 100000
F = 200000
NPW = 6256
LAST_FBASE = F - NPW
VPW = 3128
LAST_VBASE = V - VPW
CH = 128
FCHUNKS = [(k * CH, CH) for k in range(NPW // CH)] + [(NPW - NPW % CH, NPW % CH)]
VCHUNKS = [(k * CH, CH) for k in range(VPW // CH)] + [(VPW - VPW % CH, VPW % CH)]
GROUPS = NPW * 3 // 48
CORE_CHUNKS = (98, 98, 98, 97)
MAX_CORE_ROWS = 98 * 48

_mesh = plsc.VectorSubcoreMesh(core_axis_name="c", subcore_axis_name="s")


def _patterns():
    iota = lax.iota(jnp.int32, 16)
    subs = []
    for s in range(3):
        c = (iota + 16 * s) % 3
        r0 = 16 * s + iota - c
        fl = r0 // 3
        offs = jnp.where(
            c == 0, jnp.float32(0.5),
            jnp.where(c == 1, jnp.float32(1.0), jnp.float32(20.0)))
        subs.append((c, r0, fl, offs))
    return iota, subs


@functools.partial(
    pl.kernel,
    out_type=(jax.ShapeDtypeStruct((V * 8,), jnp.float32),
              jax.ShapeDtypeStruct((F * 3,), jnp.int32)),
    mesh=_mesh,
    scratch_types=[
        pltpu.VMEM((CH, 3), jnp.float32),
        pltpu.VMEM((CH * 8,), jnp.float32),
        pltpu.VMEM((CH, 3), jnp.int32),
        pltpu.VMEM((CH * 3,), jnp.int32),
    ],
    compiler_params=pltpu.CompilerParams(
        needs_layout_passes=False, use_tc_tiling_on_sc=True),
)
def _shim_in(verts_hbm, faces_hbm, v8_hbm, fidx_hbm, vstage, vbuf, fstage,
             fbuf):
    wid = lax.axis_index("s") * 2 + lax.axis_index("c")
    iota, subs = _patterns()
    lane_hi = iota >> 3
    c8 = iota & 7

    vbase = jnp.minimum(wid * VPW, LAST_VBASE)
    for off, n in VCHUNKS:
        pltpu.sync_copy(verts_hbm.at[pl.ds(vbase + off, n)],
                        vstage.at[pl.ds(0, n)])

        def vconv(u, carry):
            for t in range(4):
                row = u * 8 + t * 2 + lane_hi
                vals = plsc.load_gather(vstage, [row, c8])
                vbuf[pl.ds(u * 64 + t * 16, 16)] = vals
            return carry

        lax.fori_loop(0, n // 8, vconv, 0)
        pltpu.sync_copy(vbuf.at[pl.ds(0, n * 8)],
                        v8_hbm.at[pl.ds((vbase + off) * 8, n * 8)])

    fbase = jnp.minimum(wid * NPW, LAST_FBASE)
    for off, n in FCHUNKS:
        pltpu.sync_copy(faces_hbm.at[pl.ds(fbase + off, n)],
                        fstage.at[pl.ds(0, n)])

        def fconv(g, carry):
            for si, (c, r0, fl, offs) in enumerate(subs):
                vids = plsc.load_gather(fstage, [g * 16 + fl, c])
                fbuf[pl.ds(g * 48 + si * 16, 16)] = vids
            return carry

        lax.fori_loop(0, n // 16, fconv, 0)
        pltpu.sync_copy(fbuf.at[pl.ds(0, n * 3)],
                        fidx_hbm.at[pl.ds((fbase + off) * 3, n * 3)])


@functools.partial(
    pl.kernel,
    out_type=jax.ShapeDtypeStruct((F * 3,), jnp.float32),
    mesh=_mesh,
    scratch_types=[
        pltpu.VMEM((NPW * 3,), jnp.int32),
        pltpu.VMEM((MAX_CORE_ROWS, 8), jnp.float32),
        pltpu.VMEM((NPW * 3,), jnp.float32),
        pltpu.SemaphoreType.DMA,
    ],
    compiler_params=pltpu.CompilerParams(
        needs_layout_passes=False, use_tc_tiling_on_sc=False),
)
def _gather_core(table_hbm, fidx_hbm, out_hbm, idx_v, rows_v, out_v, sem):
    wid = lax.axis_index("s") * 2 + lax.axis_index("c")
    base = jnp.minimum(wid * NPW, LAST_FBASE) * 3
    pltpu.sync_copy(fidx_hbm.at[pl.ds(base, NPW * 3)], idx_v)
    iota, subs = _patterns()
    third = jnp.float32(1.0 / 3.0)

    chunk_base = 0
    for ngroups in CORE_CHUNKS:
        cw = ngroups * 48
        pltpu.async_copy(
            table_hbm.at[idx_v.at[pl.ds(chunk_base, cw)]],
            rows_v.at[pl.ds(0, cw)], sem).wait()

        def group(g, carry, chunk_base=chunk_base):
            gb = g * 48
            for si, (c, r0, fl, offs) in enumerate(subs):
                a = plsc.load_gather(rows_v, [gb + r0, c])
                b = plsc.load_gather(rows_v, [gb + r0 + 1, c])
                d = plsc.load_gather(rows_v, [gb + r0 + 2, c])
                out_v[pl.ds(chunk_base + gb + si * 16, 16)] = (
                    (a + b + d) * third + offs)
            return carry

        lax.fori_loop(0, ngroups, group, 0)
        chunk_base += cw

    pltpu.sync_copy(out_v, out_hbm.at[pl.ds(base, NPW * 3)])


@functools.partial(
    pl.kernel,
    out_type=jax.ShapeDtypeStruct((F, 3), jnp.float32),
    mesh=_mesh,
    scratch_types=[
        pltpu.VMEM((CH * 3,), jnp.float32),
        pltpu.VMEM((CH, 3), jnp.float32),
    ],
    compiler_params=pltpu.CompilerParams(
        needs_layout_passes=False, use_tc_tiling_on_sc=True),
)
def _shim_out(flat_hbm, out_hbm, cbuf, ostage):
    wid = lax.axis_index("s") * 2 + lax.axis_index("c")
    iota, subs = _patterns()
    fbase = jnp.minimum(wid * NPW, LAST_FBASE)
    for off, n in FCHUNKS:
        pltpu.sync_copy(flat_hbm.at[pl.ds((fbase + off) * 3, n * 3)],
                        cbuf.at[pl.ds(0, n * 3)])

        def oconv(g, carry):
            for si, (c, r0, fl, offs) in enumerate(subs):
                vals = cbuf[pl.ds(g * 48 + si * 16, 16)]
                plsc.store_scatter(ostage, [g * 16 + fl, c], vals)
            return carry

        lax.fori_loop(0, n // 16, oconv, 0)
        pltpu.sync_copy(ostage.at[pl.ds(0, n)],
                        out_hbm.at[pl.ds(fbase + off, n)])


def kernel(verts, faces_idx):
    v8_flat, fidx_flat = _shim_in(verts, faces_idx)
    centers_flat = _gather_core(v8_flat.reshape(V, 8), fidx_flat)
    return _shim_out(centers_flat)


# speedup vs baseline: 1.1875x; 1.1875x over previous
import functools

import jax
import jax.numpy as jnp
from jax import lax
from jax.experimental import pallas as pl
from jax.experimental.pallas import tpu as pltpu
from jax.experimental.pallas import tpu_sc as plsc

V = 100000
F = 200000
NPW = 6256
LAST_FBASE = F - NPW
VPW = 3128
LAST_VBASE = V - VPW
CH = 384
FCHUNKS = [(k * CH, CH) for k in range(NPW // CH)] + [(NPW - NPW % CH, NPW % CH)]
VCHUNKS = [(k * CH, CH) for k in range(VPW // CH)] + [(VPW - VPW % CH, VPW % CH)]
GROUPS = NPW * 3 // 48
CORE_CHUNKS = (98, 98, 98, 97)
MAX_CORE_ROWS = 98 * 48

_mesh = plsc.VectorSubcoreMesh(core_axis_name="c", subcore_axis_name="s")


def _patterns():
    iota = lax.iota(jnp.int32, 16)
    subs = []
    for s in range(3):
        c = (iota + 16 * s) % 3
        r0 = 16 * s + iota - c
        fl = r0 // 3
        offs = jnp.where(
            c == 0, jnp.float32(0.5),
            jnp.where(c == 1, jnp.float32(1.0), jnp.float32(20.0)))
        subs.append((c, r0, fl, offs))
    return iota, subs


@functools.partial(
    pl.kernel,
    out_type=(jax.ShapeDtypeStruct((V * 8,), jnp.float32),
              jax.ShapeDtypeStruct((F * 3,), jnp.int32)),
    mesh=_mesh,
    scratch_types=[
        pltpu.VMEM((CH, 3), jnp.float32),
        pltpu.VMEM((CH * 8,), jnp.float32),
        pltpu.VMEM((CH, 3), jnp.int32),
        pltpu.VMEM((CH * 3,), jnp.int32),
    ],
    compiler_params=pltpu.CompilerParams(
        needs_layout_passes=False, use_tc_tiling_on_sc=True),
)
def _shim_in(verts_hbm, faces_hbm, v8_hbm, fidx_hbm, vstage, vbuf, fstage,
             fbuf):
    wid = lax.axis_index("s") * 2 + lax.axis_index("c")
    iota, subs = _patterns()
    lane_hi = iota >> 3
    c8 = iota & 7

    vbase = jnp.minimum(wid * VPW, LAST_VBASE)
    for off, n in VCHUNKS:
        pltpu.sync_copy(verts_hbm.at[pl.ds(vbase + off, n)],
                        vstage.at[pl.ds(0, n)])

        def vconv(u, carry):
            for t in range(4):
                row = u * 8 + t * 2 + lane_hi
                vals = plsc.load_gather(vstage, [row, c8])
                vbuf[pl.ds(u * 64 + t * 16, 16)] = vals
            return carry

        lax.fori_loop(0, n // 8, vconv, 0)
        pltpu.sync_copy(vbuf.at[pl.ds(0, n * 8)],
                        v8_hbm.at[pl.ds((vbase + off) * 8, n * 8)])

    fbase = jnp.minimum(wid * NPW, LAST_FBASE)
    for off, n in FCHUNKS:
        pltpu.sync_copy(faces_hbm.at[pl.ds(fbase + off, n)],
                        fstage.at[pl.ds(0, n)])

        def fconv(g, carry):
            for si, (c, r0, fl, offs) in enumerate(subs):
                vids = plsc.load_gather(fstage, [g * 16 + fl, c])
                fbuf[pl.ds(g * 48 + si * 16, 16)] = vids
            return carry

        lax.fori_loop(0, n // 16, fconv, 0)
        pltpu.sync_copy(fbuf.at[pl.ds(0, n * 3)],
                        fidx_hbm.at[pl.ds((fbase + off) * 3, n * 3)])


@functools.partial(
    pl.kernel,
    out_type=jax.ShapeDtypeStruct((F * 3,), jnp.float32),
    mesh=_mesh,
    scratch_types=[
        pltpu.VMEM((NPW * 3,), jnp.int32),
        pltpu.VMEM((MAX_CORE_ROWS, 8), jnp.float32),
        pltpu.VMEM((NPW * 3,), jnp.float32),
        pltpu.SemaphoreType.DMA,
    ],
    compiler_params=pltpu.CompilerParams(
        needs_layout_passes=False, use_tc_tiling_on_sc=False),
)
def _gather_core(table_hbm, fidx_hbm, out_hbm, idx_v, rows_v, out_v, sem):
    wid = lax.axis_index("s") * 2 + lax.axis_index("c")
    base = jnp.minimum(wid * NPW, LAST_FBASE) * 3
    pltpu.sync_copy(fidx_hbm.at[pl.ds(base, NPW * 3)], idx_v)
    iota, subs = _patterns()
    third = jnp.float32(1.0 / 3.0)

    chunk_base = 0
    for ngroups in CORE_CHUNKS:
        cw = ngroups * 48
        pltpu.async_copy(
            table_hbm.at[idx_v.at[pl.ds(chunk_base, cw)]],
            rows_v.at[pl.ds(0, cw)], sem).wait()

        def group(g, carry, chunk_base=chunk_base):
            gb = g * 48
            for si, (c, r0, fl, offs) in enumerate(subs):
                a = plsc.load_gather(rows_v, [gb + r0, c])
                b = plsc.load_gather(rows_v, [gb + r0 + 1, c])
                d = plsc.load_gather(rows_v, [gb + r0 + 2, c])
                out_v[pl.ds(chunk_base + gb + si * 16, 16)] = (
                    (a + b + d) * third + offs)
            return carry

        lax.fori_loop(0, ngroups, group, 0)
        chunk_base += cw

    pltpu.sync_copy(out_v, out_hbm.at[pl.ds(base, NPW * 3)])


@functools.partial(
    pl.kernel,
    out_type=jax.ShapeDtypeStruct((F, 3), jnp.float32),
    mesh=_mesh,
    scratch_types=[
        pltpu.VMEM((CH * 3,), jnp.float32),
        pltpu.VMEM((CH, 3), jnp.float32),
    ],
    compiler_params=pltpu.CompilerParams(
        needs_layout_passes=False, use_tc_tiling_on_sc=True),
)
def _shim_out(flat_hbm, out_hbm, cbuf, ostage):
    wid = lax.axis_index("s") * 2 + lax.axis_index("c")
    iota, subs = _patterns()
    fbase = jnp.minimum(wid * NPW, LAST_FBASE)
    for off, n in FCHUNKS:
        pltpu.sync_copy(flat_hbm.at[pl.ds((fbase + off) * 3, n * 3)],
                        cbuf.at[pl.ds(0, n * 3)])

        def oconv(g, carry):
            for si, (c, r0, fl, offs) in enumerate(subs):
                vals = cbuf[pl.ds(g * 48 + si * 16, 16)]
                plsc.store_scatter(ostage, [g * 16 + fl, c], vals)
            return carry

        lax.fori_loop(0, n // 16, oconv, 0)
        pltpu.sync_copy(ostage.at[pl.ds(0, n)],
                        out_hbm.at[pl.ds(fbase + off, n)])


def kernel(verts, faces_idx):
    v8_flat, fidx_flat = _shim_in(verts, faces_idx)
    centers_flat = _gather_core(v8_flat.reshape(V, 8), fidx_flat)
    return _shim_out(centers_flat)


# speedup vs baseline: 1.2134x; 1.0219x over previous
import functools

import jax
import jax.numpy as jnp
from jax import lax
from jax.experimental import pallas as pl
from jax.experimental.pallas import tpu as pltpu
from jax.experimental.pallas import tpu_sc as plsc

V = 100000
F = 200000
NPW = 6256
LAST_FBASE = F - NPW
VPW = 3128
LAST_VBASE = V - VPW
CH = 448
FCHUNKS = [(k * CH, CH) for k in range(NPW // CH)] + [(NPW - NPW % CH, NPW % CH)]
VCHUNKS = [(k * CH, CH) for k in range(VPW // CH)] + [(VPW - VPW % CH, VPW % CH)]
GROUPS = NPW * 3 // 48
CORE_CHUNKS = (98, 98, 98, 97)
MAX_CORE_ROWS = 98 * 48

_mesh = plsc.VectorSubcoreMesh(core_axis_name="c", subcore_axis_name="s")


def _patterns():
    iota = lax.iota(jnp.int32, 16)
    subs = []
    for s in range(3):
        c = (iota + 16 * s) % 3
        r0 = 16 * s + iota - c
        fl = r0 // 3
        offs = jnp.where(
            c == 0, jnp.float32(0.5),
            jnp.where(c == 1, jnp.float32(1.0), jnp.float32(20.0)))
        subs.append((c, r0, fl, offs))
    return iota, subs


@functools.partial(
    pl.kernel,
    out_type=(jax.ShapeDtypeStruct((V * 8,), jnp.float32),
              jax.ShapeDtypeStruct((F * 3,), jnp.int32)),
    mesh=_mesh,
    scratch_types=[
        pltpu.VMEM((CH, 3), jnp.float32),
        pltpu.VMEM((CH * 8,), jnp.float32),
        pltpu.VMEM((CH, 3), jnp.int32),
        pltpu.VMEM((CH * 3,), jnp.int32),
    ],
    compiler_params=pltpu.CompilerParams(
        needs_layout_passes=False, use_tc_tiling_on_sc=True),
)
def _shim_in(verts_hbm, faces_hbm, v8_hbm, fidx_hbm, vstage, vbuf, fstage,
             fbuf):
    wid = lax.axis_index("s") * 2 + lax.axis_index("c")
    iota, subs = _patterns()
    lane_hi = iota >> 3
    c8 = iota & 7

    vbase = jnp.minimum(wid * VPW, LAST_VBASE)
    for off, n in VCHUNKS:
        pltpu.sync_copy(verts_hbm.at[pl.ds(vbase + off, n)],
                        vstage.at[pl.ds(0, n)])

        def vconv(u, carry):
            for t in range(4):
                row = u * 8 + t * 2 + lane_hi
                vals = plsc.load_gather(vstage, [row, c8])
                vbuf[pl.ds(u * 64 + t * 16, 16)] = vals
            return carry

        lax.fori_loop(0, n // 8, vconv, 0)
        pltpu.sync_copy(vbuf.at[pl.ds(0, n * 8)],
                        v8_hbm.at[pl.ds((vbase + off) * 8, n * 8)])

    fbase = jnp.minimum(wid * NPW, LAST_FBASE)
    for off, n in FCHUNKS:
        pltpu.sync_copy(faces_hbm.at[pl.ds(fbase + off, n)],
                        fstage.at[pl.ds(0, n)])

        def fconv(g, carry):
            for si, (c, r0, fl, offs) in enumerate(subs):
                vids = plsc.load_gather(fstage, [g * 16 + fl, c])
                fbuf[pl.ds(g * 48 + si * 16, 16)] = vids
            return carry

        lax.fori_loop(0, n // 16, fconv, 0)
        pltpu.sync_copy(fbuf.at[pl.ds(0, n * 3)],
                        fidx_hbm.at[pl.ds((fbase + off) * 3, n * 3)])


@functools.partial(
    pl.kernel,
    out_type=jax.ShapeDtypeStruct((F * 3,), jnp.float32),
    mesh=_mesh,
    scratch_types=[
        pltpu.VMEM((NPW * 3,), jnp.int32),
        pltpu.VMEM((MAX_CORE_ROWS, 8), jnp.float32),
        pltpu.VMEM((NPW * 3,), jnp.float32),
        pltpu.SemaphoreType.DMA,
    ],
    compiler_params=pltpu.CompilerParams(
        needs_layout_passes=False, use_tc_tiling_on_sc=False),
)
def _gather_core(table_hbm, fidx_hbm, out_hbm, idx_v, rows_v, out_v, sem):
    wid = lax.axis_index("s") * 2 + lax.axis_index("c")
    base = jnp.minimum(wid * NPW, LAST_FBASE) * 3
    pltpu.sync_copy(fidx_hbm.at[pl.ds(base, NPW * 3)], idx_v)
    iota, subs = _patterns()
    third = jnp.float32(1.0 / 3.0)

    chunk_base = 0
    for ngroups in CORE_CHUNKS:
        cw = ngroups * 48
        pltpu.async_copy(
            table_hbm.at[idx_v.at[pl.ds(chunk_base, cw)]],
            rows_v.at[pl.ds(0, cw)], sem).wait()

        def group(g, carry, chunk_base=chunk_base):
            gb = g * 48
            for si, (c, r0, fl, offs) in enumerate(subs):
                a = plsc.load_gather(rows_v, [gb + r0, c])
                b = plsc.load_gather(rows_v, [gb + r0 + 1, c])
                d = plsc.load_gather(rows_v, [gb + r0 + 2, c])
                out_v[pl.ds(chunk_base + gb + si * 16, 16)] = (
                    (a + b + d) * third + offs)
            return carry

        lax.fori_loop(0, ngroups, group, 0)
        chunk_base += cw

    pltpu.sync_copy(out_v, out_hbm.at[pl.ds(base, NPW * 3)])


@functools.partial(
    pl.kernel,
    out_type=jax.ShapeDtypeStruct((F, 3), jnp.float32),
    mesh=_mesh,
    scratch_types=[
        pltpu.VMEM((CH * 3,), jnp.float32),
        pltpu.VMEM((CH, 3), jnp.float32),
    ],
    compiler_params=pltpu.CompilerParams(
        needs_layout_passes=False, use_tc_tiling_on_sc=True),
)
def _shim_out(flat_hbm, out_hbm, cbuf, ostage):
    wid = lax.axis_index("s") * 2 + lax.axis_index("c")
    iota, subs = _patterns()
    fbase = jnp.minimum(wid * NPW, LAST_FBASE)
    for off, n in FCHUNKS:
        pltpu.sync_copy(flat_hbm.at[pl.ds((fbase + off) * 3, n * 3)],
                        cbuf.at[pl.ds(0, n * 3)])

        def oconv(g, carry):
            for si, (c, r0, fl, offs) in enumerate(subs):
                vals = cbuf[pl.ds(g * 48 + si * 16, 16)]
                plsc.store_scatter(ostage, [g * 16 + fl, c], vals)
            return carry

        lax.fori_loop(0, n // 16, oconv, 0)
        pltpu.sync_copy(ostage.at[pl.ds(0, n)],
                        out_hbm.at[pl.ds(fbase + off, n)])


def kernel(verts, faces_idx):
    v8_flat, fidx_flat = _shim_in(verts, faces_idx)
    centers_flat = _gather_core(v8_flat.reshape(V, 8), fidx_flat)
    return _shim_out(centers_flat)
